# baseline (speedup 1.0000x reference)
import functools

import jax
import jax.numpy as jnp
from jax import lax
from jax.experimental import pallas as pl
from jax.experimental.pallas import tpu as pltpu
from jax.experimental.pallas import tpu_sc as plsc

BATCH = 1024
SEQ = 50
HID = 64
NITEM = 100000
SEQ_PAD = 64
NWORK = 32
ROWS_PER_W = BATCH // NWORK
BB = 256


def _probs_body(am_ref, lm_ref, item_ref, wr_ref, ur_ref, vrw_ref,
                idx_out, val_out):
    am = am_ref[...]
    lm = lm_ref[...]
    item = item_ref[...]
    wr = wr_ref[...]
    ur = ur_ref[...]
    vrw = vrw_ref[...]

    amu = lax.dot_general(am, ur, (((2,), (1,)), ((), ())),
                          preferred_element_type=jnp.float32)
    lmw = lax.dot_general(lm, wr, (((1,), (1,)), ((), ())),
                          preferred_element_type=jnp.float32)
    t = jnp.tanh(amu + lmw[:, None, :])
    s = jnp.sum(t * vrw[0][None, None, :], axis=-1)
    s = s - jnp.max(s, axis=-1, keepdims=True)
    e = jnp.exp(s)
    p = e / jnp.sum(e, axis=-1, keepdims=True)

    it = item[:, :SEQ]
    eq = it[:, :, None] == it[:, None, :]
    comb = jnp.sum(jnp.where(eq, p[:, None, :], 0.0), axis=-1)
    qlt = (jnp.arange(SEQ)[:, None] > jnp.arange(SEQ)[None, :])[None]
    firsti = jnp.where(
        jnp.sum(jnp.where(eq & qlt, 1, 0), axis=-1) == 0, 1, 0)

    lane = (jnp.arange(SEQ_PAD, dtype=jnp.int32) % 16)[None, :]
    pad_cols = SEQ_PAD - SEQ
    first_p = jnp.pad(firsti, ((0, 0), (0, pad_cols))) > 0
    comb_p = jnp.pad(comb, ((0, 0), (0, pad_cols)))
    it_p = jnp.pad(it, ((0, 0), (0, pad_cols)))
    idx_out[...] = jnp.where(first_p, it_p, NITEM + lane).astype(jnp.int32)
    val_out[...] = jnp.where(first_p, comb_p, 0.0)


def _compute_scatter_args(all_memory, last_memory, seq_item, Wr, Ur, Vr_w):
    grid = BATCH // BB
    return pl.pallas_call(
        _probs_body,
        grid=(grid,),
        in_specs=[
            pl.BlockSpec((BB, SEQ, HID), lambda i: (i, 0, 0)),
            pl.BlockSpec((BB, HID), lambda i: (i, 0)),
            pl.BlockSpec((BB, SEQ_PAD), lambda i: (i, 0)),
            pl.BlockSpec((HID, HID), lambda i: (0, 0)),
            pl.BlockSpec((HID, HID), lambda i: (0, 0)),
            pl.BlockSpec((1, HID), lambda i: (0, 0)),
        ],
        out_specs=[
            pl.BlockSpec((BB, SEQ_PAD), lambda i: (i, 0)),
            pl.BlockSpec((BB, SEQ_PAD), lambda i: (i, 0)),
        ],
        out_shape=[
            jax.ShapeDtypeStruct((BATCH, SEQ_PAD), jnp.int32),
            jax.ShapeDtypeStruct((BATCH, SEQ_PAD), jnp.float32),
        ],
    )(all_memory, last_memory, seq_item, Wr, Ur, Vr_w)


@functools.cache
def _make_scatter_kernel():
    return pl.kernel(
        _scatter_body,
        out_type=jax.ShapeDtypeStruct((BATCH, NITEM), jnp.float32),
        mesh=plsc.VectorSubcoreMesh(core_axis_name="c", subcore_axis_name="s",
                                    num_cores=2, num_subcores=16),
        compiler_params=pltpu.CompilerParams(needs_layout_passes=False,
                                             use_tc_tiling_on_sc=False),
        scratch_types=[
            pltpu.VMEM((NITEM + 16,), jnp.float32),
            pltpu.VMEM((ROWS_PER_W * SEQ_PAD,), jnp.int32),
            pltpu.VMEM((ROWS_PER_W * SEQ_PAD,), jnp.float32),
        ],
    )


def _scatter_body(idx_hbm, val_hbm, out_hbm, row_buf, idx_v, val_v):
    wid = lax.axis_index("s") * 2 + lax.axis_index("c")
    base = wid * ROWS_PER_W

    pltpu.sync_copy(idx_hbm.at[pl.ds(base * SEQ_PAD, ROWS_PER_W * SEQ_PAD)],
                    idx_v)
    pltpu.sync_copy(val_hbm.at[pl.ds(base * SEQ_PAD, ROWS_PER_W * SEQ_PAD)],
                    val_v)

    zeros16 = jnp.zeros((16,), jnp.float32)

    def zinit(i, carry):
        row_buf[pl.ds(i * 16, 16)] = zeros16
        return carry

    lax.fori_loop(0, (NITEM + 16) // 16, zinit, 0)

    def do_row(r, carry):
        off = r * SEQ_PAD
        for k in range(SEQ_PAD // 16):
            idx = idx_v[pl.ds(off + k * 16, 16)]
            val = val_v[pl.ds(off + k * 16, 16)]
            plsc.addupdate_scatter(row_buf, [idx], val)
        pltpu.sync_copy(row_buf.at[pl.ds(0, NITEM)], out_hbm.at[base + r])
        for k in range(SEQ_PAD // 16):
            idx = idx_v[pl.ds(off + k * 16, 16)]
            plsc.store_scatter(row_buf, [idx], zeros16)
        return carry

    lax.fori_loop(0, ROWS_PER_W, do_row, 0)


def kernel(all_memory, last_memory, seq_item, Wr, Ur, Vr_w, Vr_b):
    del Vr_b
    seq_item = seq_item.astype(jnp.int32)
    item_pad = jnp.pad(seq_item, ((0, 0), (0, SEQ_PAD - SEQ)))
    idx, val = _compute_scatter_args(
        all_memory, last_memory, item_pad, Wr, Ur, Vr_w)
    return _make_scatter_kernel()(idx.reshape(-1), val.reshape(-1))
